# baseline (speedup 1.0000x reference)
import functools

import jax
import jax.numpy as jnp
from jax import lax
from jax.experimental import pallas as pl
from jax.experimental.pallas import tpu as pltpu
from jax.experimental.pallas import tpu_sc as plsc

B, C_IN, C_OUT, L, K = 4, 16, 16, 4096, 4
TR = 256
NT = L // TR
BIG = 3.0e38
D = 128

_NC, _NS = 2, 16
_NW = _NC * _NS
_TPW = (B * L) // _NW
_IPW = _TPW * K
_ICH = 128
_NCH = _IPW // _ICH
_GCH = 4
_TCH = (_GCH * _ICH) // K


def _topk_body(x_rows_ref, x_all_ref, idx_ref):
    b = pl.program_id(0)
    xr = x_rows_ref[0]
    xa = x_all_ref[0]
    dot = lax.dot_general(xr, xa, (((0,), (0,)), ((), ())),
                          preferred_element_type=jnp.float32)
    na = jnp.sum(xa * xa, axis=0, keepdims=True)
    ones = jnp.ones((C_IN, 1), dtype=jnp.float32)
    nr = lax.dot_general(xr * xr, ones, (((0,), (0,)), ((), ())),
                         preferred_element_type=jnp.float32)
    d = (na + nr) - 2.0 * dot
    d = jnp.maximum(d, 0.0)

    col = lax.broadcasted_iota(jnp.int32, (TR, L), 1)
    iks = []
    for k in range(K):
        m = jnp.min(d, axis=1, keepdims=True)
        ik = jnp.min(jnp.where(d == m, col, L), axis=1, keepdims=True)
        iks.append(ik)
        if k < K - 1:
            d = jnp.where(col == ik, BIG, d)

    col4 = lax.broadcasted_iota(jnp.int32, (TR, K), 1)
    idxs = jnp.where(col4 == 0, iks[0],
                     jnp.where(col4 == 1, iks[1],
                               jnp.where(col4 == 2, iks[2], iks[3])))
    idx_ref[0] = idxs + b * L


def _topk(x):
    return pl.pallas_call(
        _topk_body,
        grid=(B, NT),
        in_specs=[
            pl.BlockSpec((1, C_IN, TR), lambda b, i: (b, 0, i)),
            pl.BlockSpec((1, C_IN, L), lambda b, i: (b, 0, 0)),
        ],
        out_specs=pl.BlockSpec((1, TR, K), lambda b, i: (b * NT + i, 0, 0)),
        out_shape=jax.ShapeDtypeStruct((B * NT, TR, K), jnp.int32),
    )(x, x)


def _ytable_body(x_ref, w_ref, bias_ref, out_ref):
    xa = x_ref[0]
    yt = lax.dot_general(xa, w_ref[...], (((0,), (0,)), ((), ())),
                         preferred_element_type=jnp.float32)
    out_ref[0] = yt + bias_ref[...][None, :]


def _ytable(x, W, bias):
    Wp = jnp.zeros((C_IN, D), jnp.float32)
    Wp = Wp.at[:, : K * C_OUT].set(W.transpose(1, 2, 0).reshape(C_IN, K * C_OUT))
    bp = jnp.zeros((D,), jnp.float32)
    bp = bp.at[: K * C_OUT].set(jnp.tile(bias * (1.0 / K), K))
    return pl.pallas_call(
        _ytable_body,
        grid=(B,),
        in_specs=[
            pl.BlockSpec((1, C_IN, L), lambda b: (b, 0, 0)),
            pl.BlockSpec((C_IN, D), lambda b: (0, 0)),
            pl.BlockSpec((D,), lambda b: (0,)),
        ],
        out_specs=pl.BlockSpec((1, L, D), lambda b: (b, 0, 0)),
        out_shape=jax.ShapeDtypeStruct((B, L, D), jnp.float32),
    )(x, Wp, bp)


def _gather_body(table_hbm, idx_hbm, out_hbm, idx_v, rows_v, out_v, sem):
    c = lax.axis_index("c")
    s = lax.axis_index("s")
    wid = s * _NC + c
    pltpu.sync_copy(idx_hbm.at[pl.ds(wid * _NCH, _NCH)], idx_v)
    for g in range(_NCH // _GCH):
        copies = [
            pltpu.async_copy(table_hbm.at[idx_v.at[g * _GCH + j]],
                             rows_v.at[pl.ds(j * _ICH, _ICH)], sem)
            for j in range(_GCH)
        ]
        for cp in copies:
            cp.wait()

        def body(t, carry):
            base = t * K
            r = ((rows_v[base, pl.ds(0, 16)]
                  + rows_v[base + 1, pl.ds(16, 16)])
                 + (rows_v[base + 2, pl.ds(32, 16)]
                    + rows_v[base + 3, pl.ds(48, 16)]))
            out_v[g * _TCH + t] = r
            return carry

        lax.fori_loop(0, _TCH, body, 0)
    pltpu.sync_copy(out_v, out_hbm.at[pl.ds(wid * _TPW, _TPW)])


def _gather_sum(table, idx_flat2d):
    mesh = plsc.VectorSubcoreMesh(core_axis_name="c", subcore_axis_name="s")
    run = functools.partial(
        pl.kernel,
        out_type=jax.ShapeDtypeStruct((B * L, C_OUT), jnp.float32),
        mesh=mesh,
        scratch_types=[
            pltpu.VMEM((_NCH, _ICH), jnp.int32),
            pltpu.VMEM((_GCH * _ICH, D), jnp.float32),
            pltpu.VMEM((_TPW, C_OUT), jnp.float32),
            pltpu.SemaphoreType.DMA,
        ],
        compiler_params=pltpu.CompilerParams(use_tc_tiling_on_sc=False),
    )(_gather_body)
    return run(table, idx_flat2d)


def kernel(x, W, b):
    idx = _topk(x)
    yt = _ytable(x, W, b)
    table = yt.reshape(B * L, D)
    idx2d = idx.reshape((B * L * K) // _ICH, _ICH)
    out_flat = _gather_sum(table, idx2d)
    return out_flat.reshape(B, L, C_OUT).transpose(0, 2, 1)


# speedup vs baseline: 1.3459x; 1.3459x over previous
import functools

import jax
import jax.numpy as jnp
from jax import lax
from jax.experimental import pallas as pl
from jax.experimental.pallas import tpu as pltpu
from jax.experimental.pallas import tpu_sc as plsc

B, C_IN, C_OUT, L, K = 4, 16, 16, 4096, 4
TR = 512
NT = L // TR
BIG = 3.0e38
D = 128
CA = 24

_NC, _NS = 2, 16
_NW = _NC * _NS
_TPW = (B * L) // _NW
_IPW = _TPW * K
_ICH = 128
_NCH = _IPW // _ICH
_GCH = 4
_TCH = (_GCH * _ICH) // K


def _topk_body(x_rows_ref, x_all_ref, w_ref, bias_ref, idx_ref, yt_ref):
    b = pl.program_id(0)
    xr = x_rows_ref[0]
    xa = x_all_ref[0]
    nr = jnp.sum(xr * xr, axis=0, keepdims=True)
    na = jnp.sum(xa * xa, axis=0, keepdims=True)
    one_r = jnp.ones((1, TR), jnp.float32)
    one_a = jnp.ones((1, L), jnp.float32)
    pad_r = jnp.zeros((CA - C_IN - 2, TR), jnp.float32)
    pad_a = jnp.zeros((CA - C_IN - 2, L), jnp.float32)
    lhs = jnp.concatenate([-2.0 * xr, nr, one_r, pad_r], axis=0)
    rhs = jnp.concatenate([xa, one_a, na, pad_a], axis=0)
    d = lax.dot_general(lhs, rhs, (((0,), (0,)), ((), ())),
                        preferred_element_type=jnp.float32)
    d = jnp.maximum(d, 0.0)

    yt_ref[0] = (lax.dot_general(xr, w_ref[...], (((0,), (0,)), ((), ())),
                                 preferred_element_type=jnp.float32)
                 + bias_ref[...][None, :])

    colf = lax.broadcasted_iota(jnp.int32, (TR, L), 1).astype(jnp.float32)
    iks = []
    for k in range(K):
        m = jnp.min(d, axis=1, keepdims=True)
        ikf = jnp.min(jnp.where(d == m, colf, float(L)),
                      axis=1, keepdims=True)
        iks.append(ikf)
        if k < K - 1:
            d = jnp.where(colf == ikf, BIG, d)

    col4 = lax.broadcasted_iota(jnp.int32, (TR, K), 1)
    idxs = jnp.where(col4 == 0, iks[0],
                     jnp.where(col4 == 1, iks[1],
                               jnp.where(col4 == 2, iks[2], iks[3])))
    idx_ref[0] = idxs.astype(jnp.int32) + b * L


def _topk_ytable(x, W, bias):
    Wp = jnp.zeros((C_IN, D), jnp.float32)
    Wp = Wp.at[:, : K * C_OUT].set(W.transpose(1, 2, 0).reshape(C_IN, K * C_OUT))
    bp = jnp.zeros((D,), jnp.float32)
    bp = bp.at[: K * C_OUT].set(jnp.tile(bias * (1.0 / K), K))
    return pl.pallas_call(
        _topk_body,
        grid=(B, NT),
        in_specs=[
            pl.BlockSpec((1, C_IN, TR), lambda b, i: (b, 0, i)),
            pl.BlockSpec((1, C_IN, L), lambda b, i: (b, 0, 0)),
            pl.BlockSpec((C_IN, D), lambda b, i: (0, 0)),
            pl.BlockSpec((D,), lambda b, i: (0,)),
        ],
        out_specs=[
            pl.BlockSpec((1, TR, K), lambda b, i: (b * NT + i, 0, 0)),
            pl.BlockSpec((1, TR, D), lambda b, i: (b * NT + i, 0, 0)),
        ],
        out_shape=[
            jax.ShapeDtypeStruct((B * NT, TR, K), jnp.int32),
            jax.ShapeDtypeStruct((B * NT, TR, D), jnp.float32),
        ],
    )(x, x, Wp, bp)


def _gather_body(table_hbm, idx_hbm, out_hbm, idx_v, rows_v, out_v, sem):
    c = lax.axis_index("c")
    s = lax.axis_index("s")
    wid = s * _NC + c
    pltpu.sync_copy(idx_hbm.at[pl.ds(wid * _NCH, _NCH)], idx_v)
    for g in range(_NCH // _GCH):
        copies = [
            pltpu.async_copy(table_hbm.at[idx_v.at[g * _GCH + j]],
                             rows_v.at[pl.ds(j * _ICH, _ICH)], sem)
            for j in range(_GCH)
        ]
        for cp in copies:
            cp.wait()

        def body(t, carry):
            base = t * K
            r = ((rows_v[base, pl.ds(0, 16)]
                  + rows_v[base + 1, pl.ds(16, 16)])
                 + (rows_v[base + 2, pl.ds(32, 16)]
                    + rows_v[base + 3, pl.ds(48, 16)]))
            out_v[g * _TCH + t] = r
            return carry

        lax.fori_loop(0, _TCH, body, 0)
    pltpu.sync_copy(out_v, out_hbm.at[pl.ds(wid * _TPW, _TPW)])


def _gather_sum(table, idx_flat2d):
    mesh = plsc.VectorSubcoreMesh(core_axis_name="c", subcore_axis_name="s")
    run = functools.partial(
        pl.kernel,
        out_type=jax.ShapeDtypeStruct((B * L, C_OUT), jnp.float32),
        mesh=mesh,
        scratch_types=[
            pltpu.VMEM((_NCH, _ICH), jnp.int32),
            pltpu.VMEM((_GCH * _ICH, D), jnp.float32),
            pltpu.VMEM((_TPW, C_OUT), jnp.float32),
            pltpu.SemaphoreType.DMA,
        ],
        compiler_params=pltpu.CompilerParams(use_tc_tiling_on_sc=False),
    )(_gather_body)
    return run(table, idx_flat2d)


def kernel(x, W, b):
    idx, yt = _topk_ytable(x, W, b)
    table = yt.reshape(B * L, D)
    idx2d = idx.reshape((B * L * K) // _ICH, _ICH)
    out_flat = _gather_sum(table, idx2d)
    return out_flat.reshape(B, L, C_OUT).transpose(0, 2, 1)
